# baseline (speedup 1.0000x reference)
import functools
import numpy as np
import jax
import jax.numpy as jnp
from jax import lax
from jax.experimental import pallas as pl
from jax.experimental.pallas import tpu as pltpu
from jax.experimental.pallas import tpu_sc as plsc

BATCH = 16384
NUM_FIELDS = 26
EMBED_DIM = 16
FIELD_SIZE = 100000

NC, NS = 2, 16
NW = NC * NS
TOTAL = BATCH * NUM_FIELDS
ROWS = TOTAL // 128
ROWS_W = ROWS // NW
PERIOD = 13
GROUPS = ROWS_W // PERIOD
GROUP_IDX = PERIOD * 128

_offs_np = (FIELD_SIZE * (np.arange(PERIOD * 128) % NUM_FIELDS)).astype(np.int32)
_OFFS_PATTERN = _offs_np.reshape(PERIOD, 128)

_mesh = plsc.VectorSubcoreMesh(core_axis_name="c", subcore_axis_name="s")


@functools.partial(
    pl.kernel,
    out_type=jax.ShapeDtypeStruct((TOTAL, EMBED_DIM), jnp.float32),
    mesh=_mesh,
    scratch_types=[
        pltpu.VMEM((ROWS_W, 128), jnp.int32),
        pltpu.VMEM((PERIOD, 128), jnp.int32),
        pltpu.VMEM((2, GROUP_IDX, EMBED_DIM), jnp.float32),
        pltpu.SemaphoreType.DMA,
    ],
    compiler_params=pltpu.CompilerParams(use_tc_tiling_on_sc=False),
)
def _emb_lookup(x_hbm, offs_hbm, table_hbm, out_hbm, idx_v, offs_v, rows_v, sem):
    wid = lax.axis_index("s") * NC + lax.axis_index("c")
    row_base = wid * ROWS_W

    pltpu.sync_copy(x_hbm.at[pl.ds(row_base, ROWS_W)], idx_v)
    pltpu.sync_copy(offs_hbm, offs_v)

    def add_offsets(g, carry):
        for j in range(PERIOD):
            row = g * PERIOD + j
            for c in range(8):
                sl = pl.ds(c * 16, 16)
                idx_v[row, sl] = idx_v[row, sl] + offs_v[j, sl]
        return carry

    lax.fori_loop(0, GROUPS, add_offsets, 0)

    def fire(g, buf):
        return [
            pltpu.async_copy(
                table_hbm.at[idx_v.at[g * PERIOD + j]],
                rows_v.at[buf, pl.ds(j * 128, 128)],
                sem,
            )
            for j in range(PERIOD)
        ]

    out_base = wid * ROWS_W * 128
    pending = fire(0, 0)
    for g in range(1, GROUPS):
        nxt = fire(g, g % 2)
        for cp in pending:
            cp.wait()
        pltpu.sync_copy(
            rows_v.at[(g - 1) % 2],
            out_hbm.at[pl.ds(out_base + (g - 1) * GROUP_IDX, GROUP_IDX)],
        )
        pending = nxt
    for cp in pending:
        cp.wait()
    pltpu.sync_copy(
        rows_v.at[(GROUPS - 1) % 2],
        out_hbm.at[pl.ds(out_base + (GROUPS - 1) * GROUP_IDX, GROUP_IDX)],
    )


def kernel(x, table):
    x2 = x.reshape(ROWS, 128)
    offs = jnp.asarray(_OFFS_PATTERN)
    out = _emb_lookup(x2, offs, table)
    return out.reshape(BATCH, NUM_FIELDS, EMBED_DIM)


# speedup vs baseline: 1.2816x; 1.2816x over previous
import functools
import jax
import jax.numpy as jnp
from jax import lax
from jax.experimental import pallas as pl
from jax.experimental.pallas import tpu as pltpu
from jax.experimental.pallas import tpu_sc as plsc

BATCH = 16384
NUM_FIELDS = 26
EMBED_DIM = 16
FIELD_SIZE = 100000

NC, NS = 2, 16
NW = NC * NS
NBLK = NUM_FIELDS * (BATCH // 128)
BLK_W = NBLK // NW
TBLK = (2600000 * EMBED_DIM) // 128

_mesh = plsc.VectorSubcoreMesh(core_axis_name="c", subcore_axis_name="s")


@functools.partial(
    pl.kernel,
    out_type=jax.ShapeDtypeStruct((NUM_FIELDS, 2, 128, 8, 128), jnp.float32),
    mesh=_mesh,
    scratch_types=[
        pltpu.VMEM((BLK_W, 128), jnp.int32),
        pltpu.VMEM((BLK_W, 128), jnp.int32),
        pltpu.VMEM((2, 128, 128), jnp.float32),
        pltpu.VMEM((2, 2, 8, 128), jnp.float32),
        pltpu.SemaphoreType.DMA,
        pltpu.SemaphoreType.DMA,
    ],
    compiler_params=pltpu.CompilerParams(
        use_tc_tiling_on_sc=True, needs_layout_passes=False
    ),
)
def _emb_lookup(x_hbm, table_hbm, out_hbm, idx_v, sub_v, blk_v, tile_v, gsem, wsem):
    wid = lax.axis_index("s") * NC + lax.axis_index("c")
    g0 = wid * BLK_W

    pltpu.sync_copy(x_hbm.at[pl.ds(g0, BLK_W)], idx_v)

    def prep(r, carry):
        off = FIELD_SIZE * ((g0 + r) // 128)
        for c in range(8):
            sl = pl.ds(c * 16, 16)
            full = idx_v[r, sl] + off
            idx_v[r, sl] = full >> 3
            sub_v[r, sl] = full & 7
        return carry

    lax.fori_loop(0, BLK_W, prep, 0)

    bvec = lax.broadcasted_iota(jnp.int32, (16,), 0)

    def gather_of(r):
        return pltpu.make_async_copy(
            table_hbm.at[idx_v.at[r]], blk_v.at[r & 1], gsem
        )

    def write_of(r, dg):
        g = g0 + r
        return pltpu.make_async_copy(
            tile_v.at[r & 1, dg], out_hbm.at[g // 128, dg, g % 128], wsem
        )

    gather_of(0).start()

    def body(r, carry):
        @pl.when(r + 1 < BLK_W)
        def _():
            gather_of(r + 1).start()

        @pl.when(r >= 2)
        def _():
            write_of(r - 2, 0).wait()
            write_of(r - 2, 1).wait()

        gather_of(r).wait()

        buf = r & 1
        for k in range(8):
            row = bvec + k * 16
            sv = sub_v[r, pl.ds(k * 16, 16)]
            col0 = sv << 4
            for dgdd in range(16):
                val = plsc.load_gather(blk_v.at[buf], [row, col0 + dgdd])
                tile_v[buf, dgdd // 8, dgdd % 8, pl.ds(k * 16, 16)] = val

        write_of(r, 0).start()
        write_of(r, 1).start()
        return carry

    lax.fori_loop(0, BLK_W, body, 0)
    write_of(BLK_W - 2, 0).wait()
    write_of(BLK_W - 2, 1).wait()
    write_of(BLK_W - 1, 0).wait()
    write_of(BLK_W - 1, 1).wait()


def kernel(x, table):
    x2 = x.T.reshape(NBLK, 128)
    tableB = table.reshape(TBLK, 128)
    out5 = _emb_lookup(x2, tableB)
    return out5.transpose(2, 4, 0, 1, 3).reshape(BATCH, NUM_FIELDS, EMBED_DIM)


# speedup vs baseline: 1.4475x; 1.1294x over previous
import functools
import jax
import jax.numpy as jnp
from jax import lax
from jax.experimental import pallas as pl
from jax.experimental.pallas import tpu as pltpu
from jax.experimental.pallas import tpu_sc as plsc

BATCH = 16384
NUM_FIELDS = 26
EMBED_DIM = 16
FIELD_SIZE = 100000
NUM_EMB = NUM_FIELDS * FIELD_SIZE

NC, NS = 2, 16
NW = NC * NS
NBLK = NUM_FIELDS * (BATCH // 128)
BLK_W = NBLK // NW
TBLK = NUM_EMB * EMBED_DIM // 128

CW = 512
NMAIN = (NUM_EMB // CW) * CW
NCH = NMAIN // CW
CH_BASE = NCH // NW
CH_EXTRA = NCH - CH_BASE * NW

_mesh = plsc.VectorSubcoreMesh(core_axis_name="c", subcore_axis_name="s")


@functools.partial(
    pl.kernel,
    out_type=jax.ShapeDtypeStruct((TBLK, 128), jnp.float32),
    mesh=_mesh,
    scratch_types=[
        pltpu.VMEM((2, EMBED_DIM, CW), jnp.float32),
        pltpu.VMEM((2, CW // 8, 128), jnp.float32),
        pltpu.VMEM((EMBED_DIM, 128), jnp.float32),
        pltpu.SemaphoreType.DMA,
        pltpu.SemaphoreType.DMA,
    ],
    compiler_params=pltpu.CompilerParams(
        use_tc_tiling_on_sc=True, needs_layout_passes=False
    ),
)
def _relayout(tt_hbm, tail_hbm, out_hbm, stage_v, rows_v, tail_v, gsem, wsem):
    wid = lax.axis_index("s") * NC + lax.axis_index("c")
    c0 = wid * CH_BASE + jnp.minimum(wid, CH_EXTRA)
    nch = CH_BASE + (wid < CH_EXTRA).astype(jnp.int32)

    dvec = lax.broadcasted_iota(jnp.int32, (16,), 0)

    def stage_of(i):
        return pltpu.make_async_copy(
            tt_hbm.at[:, pl.ds((c0 + i) * CW, CW)], stage_v.at[i & 1], gsem
        )

    def write_of(i):
        return pltpu.make_async_copy(
            rows_v.at[i & 1], out_hbm.at[pl.ds((c0 + i) * (CW // 8), CW // 8)],
            wsem,
        )

    stage_of(0).start()

    def body(i, carry):
        @pl.when(i + 1 < nch)
        def _():
            stage_of(i + 1).start()

        @pl.when(i >= 2)
        def _():
            write_of(i - 2).wait()

        stage_of(i).wait()
        buf = i & 1

        def inner(j, carry2):
            for u in range(16):
                col = jnp.broadcast_to(j * 16 + u, (16,))
                val = plsc.load_gather(stage_v.at[buf], [dvec, col])
                rows_v[buf, j * 2 + u // 8, pl.ds((u % 8) * 16, 16)] = val
            return carry2

        lax.fori_loop(0, CW // 16, inner, 0)
        write_of(i).start()
        return carry

    lax.fori_loop(0, nch, body, 0)
    write_of(nch - 2).wait()
    write_of(nch - 1).wait()

    @pl.when(wid == NW - 1)
    def _():
        pltpu.sync_copy(tail_hbm, tail_v)
        for r in range(64):
            col = jnp.broadcast_to(jnp.int32(r), (16,))
            val = plsc.load_gather(tail_v, [dvec, col])
            rows_v[0, r // 8, pl.ds((r % 8) * 16, 16)] = val
        pltpu.sync_copy(
            rows_v.at[0, pl.ds(0, 8)], out_hbm.at[pl.ds(NMAIN // 8, 8)]
        )


@functools.partial(
    pl.kernel,
    out_type=jax.ShapeDtypeStruct((NUM_FIELDS, 2, 128, 8, 128), jnp.float32),
    mesh=_mesh,
    scratch_types=[
        pltpu.VMEM((BLK_W, 128), jnp.int32),
        pltpu.VMEM((BLK_W, 128), jnp.int32),
        pltpu.VMEM((2, 128, 128), jnp.float32),
        pltpu.VMEM((2, 2, 8, 128), jnp.float32),
        pltpu.SemaphoreType.DMA,
        pltpu.SemaphoreType.DMA,
    ],
    compiler_params=pltpu.CompilerParams(
        use_tc_tiling_on_sc=True, needs_layout_passes=False
    ),
)
def _emb_lookup(x_hbm, table_hbm, out_hbm, idx_v, sub_v, blk_v, tile_v, gsem, wsem):
    wid = lax.axis_index("s") * NC + lax.axis_index("c")
    g0 = wid * BLK_W

    pltpu.sync_copy(x_hbm.at[pl.ds(g0, BLK_W)], idx_v)

    def prep(r, carry):
        off = FIELD_SIZE * ((g0 + r) // 128)
        for c in range(8):
            sl = pl.ds(c * 16, 16)
            full = idx_v[r, sl] + off
            idx_v[r, sl] = full >> 3
            sub_v[r, sl] = full & 7
        return carry

    lax.fori_loop(0, BLK_W, prep, 0)

    bvec = lax.broadcasted_iota(jnp.int32, (16,), 0)

    def gather_of(r):
        return pltpu.make_async_copy(
            table_hbm.at[idx_v.at[r]], blk_v.at[r & 1], gsem
        )

    def write_of(r, dg):
        g = g0 + r
        return pltpu.make_async_copy(
            tile_v.at[r & 1, dg], out_hbm.at[g // 128, dg, g % 128], wsem
        )

    gather_of(0).start()

    def body(r, carry):
        @pl.when(r + 1 < BLK_W)
        def _():
            gather_of(r + 1).start()

        @pl.when(r >= 2)
        def _():
            write_of(r - 2, 0).wait()
            write_of(r - 2, 1).wait()

        gather_of(r).wait()

        buf = r & 1
        for k in range(8):
            row = bvec + k * 16
            sv = sub_v[r, pl.ds(k * 16, 16)]
            col0 = sv << 4
            for dgdd in range(16):
                val = plsc.load_gather(blk_v.at[buf], [row, col0 + dgdd])
                tile_v[buf, dgdd // 8, dgdd % 8, pl.ds(k * 16, 16)] = val

        write_of(r, 0).start()
        write_of(r, 1).start()
        return carry

    lax.fori_loop(0, BLK_W, body, 0)
    write_of(BLK_W - 2, 0).wait()
    write_of(BLK_W - 2, 1).wait()
    write_of(BLK_W - 1, 0).wait()
    write_of(BLK_W - 1, 1).wait()


def kernel(x, table):
    tt = table.T
    tail = jnp.pad(tt[:, NMAIN:], ((0, 0), (0, 128 - (NUM_EMB - NMAIN))))
    tableB = _relayout(tt, tail)
    x2 = x.T.reshape(NBLK, 128)
    out5 = _emb_lookup(x2, tableB)
    return out5.transpose(2, 4, 0, 1, 3).reshape(BATCH, NUM_FIELDS, EMBED_DIM)
